# baseline (speedup 1.0000x reference)
import functools
import math

import jax
import jax.numpy as jnp
from jax.experimental import pallas as pl
from jax.experimental.pallas import tpu as pltpu

_EPS = 0.05
_ITERS = 200


def _emd_kernel(pc1_ref, pc2t_ref, out_ref, k_ref, *, n_pts, n_iters):
    eps = jnp.float32(_EPS)
    w = jnp.float32(1.0 / n_pts)

    a = pc1_ref[0]
    bt = pc2t_ref[0]

    d2 = jnp.zeros((n_pts, n_pts), jnp.float32)
    for k in range(3):
        diff = a[:, k : k + 1] - bt[k : k + 1, :]
        d2 = d2 + diff * diff
    negc = -jnp.sqrt(d2 + jnp.float32(1e-12)) / eps

    alpha = -jnp.max(negc, axis=1, keepdims=True)
    k_ref[...] = jnp.exp(negc + alpha)

    floor = jnp.float32(1e-35)
    cap = jnp.float32(1e30)

    def half_steps(v):
        kmat = k_ref[...]
        t1 = jnp.sum(kmat * v, axis=1, keepdims=True)
        u = jnp.minimum(w / jnp.maximum(t1, floor), cap)
        t2 = jnp.sum(kmat * u, axis=0, keepdims=True)
        v = jnp.minimum(w / jnp.maximum(t2, floor), cap)
        return u, v

    v0 = jnp.full((1, n_pts), w, jnp.float32)
    v = jax.lax.fori_loop(0, n_iters - 1, lambda _, v: half_steps(v)[1], v0)
    u, v = half_steps(v)

    kmat = k_ref[...]
    kc = kmat * (eps * (alpha - jnp.log(jnp.maximum(kmat, jnp.float32(1e-37)))))
    total = jnp.sum(u * jnp.sum(kc * v, axis=1, keepdims=True))
    out_ref[...] = jnp.full(out_ref.shape, total, jnp.float32)


def kernel(pc1, pc2):
    b, n, _ = pc1.shape
    pc2t = pc2.transpose(0, 2, 1)
    per_batch = pl.pallas_call(
        functools.partial(_emd_kernel, n_pts=n, n_iters=_ITERS),
        grid=(b,),
        in_specs=[
            pl.BlockSpec((1, n, 3), lambda i: (i, 0, 0)),
            pl.BlockSpec((1, 3, n), lambda i: (i, 0, 0)),
        ],
        out_specs=pl.BlockSpec((1, 1, 128), lambda i: (i, 0, 0)),
        out_shape=jax.ShapeDtypeStruct((b, 1, 128), jnp.float32),
        scratch_shapes=[pltpu.VMEM((n, n), jnp.float32)],
        compiler_params=pltpu.CompilerParams(
            dimension_semantics=("parallel",),
            vmem_limit_bytes=100 * 1024 * 1024,
        ),
        name="sinkhorn_emd",
    )(pc1, pc2t)
    return jnp.sum(per_batch[:, 0, 0])


# speedup vs baseline: 1.0205x; 1.0205x over previous
import functools
import math

import jax
import jax.numpy as jnp
from jax.experimental import pallas as pl
from jax.experimental.pallas import tpu as pltpu

_EPS = 0.05
_ITERS = 200


def _emd_kernel(pc1_ref, pc2t_ref, out_ref, k_ref, *, n_pts, n_iters):
    eps = jnp.float32(_EPS)
    w = jnp.float32(1.0 / n_pts)

    a = pc1_ref[0]
    bt = pc2t_ref[0]

    d2 = jnp.zeros((n_pts, n_pts), jnp.float32)
    for k in range(3):
        diff = a[:, k : k + 1] - bt[k : k + 1, :]
        d2 = d2 + diff * diff
    negc = -jnp.sqrt(d2 + jnp.float32(1e-12)) / eps

    alpha = -jnp.max(negc, axis=1, keepdims=True)
    k_ref[...] = jnp.exp(negc + alpha)

    floor = jnp.float32(1e-35)
    cap = jnp.float32(1e30)

    def body(_, carry):
        u, v = carry
        kmat = k_ref[...]
        t1 = jnp.sum(kmat * v, axis=1, keepdims=True)
        u = jnp.minimum(w / jnp.maximum(t1, floor), cap)
        t2 = jnp.sum(kmat * u, axis=0, keepdims=True)
        v = jnp.minimum(w / jnp.maximum(t2, floor), cap)
        return u, v

    u0 = jnp.full((n_pts, 1), w, jnp.float32)
    v0 = jnp.full((1, n_pts), w, jnp.float32)
    u, v = jax.lax.fori_loop(0, n_iters, body, (u0, v0))

    kmat = k_ref[...]
    kc = kmat * (eps * (alpha - jnp.log(jnp.maximum(kmat, jnp.float32(1e-37)))))
    total = jnp.sum(u * jnp.sum(kc * v, axis=1, keepdims=True))
    out_ref[...] = jnp.full(out_ref.shape, total, jnp.float32)


def kernel(pc1, pc2):
    b, n, _ = pc1.shape
    pc2t = pc2.transpose(0, 2, 1)
    per_batch = pl.pallas_call(
        functools.partial(_emd_kernel, n_pts=n, n_iters=_ITERS),
        grid=(b,),
        in_specs=[
            pl.BlockSpec((1, n, 3), lambda i: (i, 0, 0)),
            pl.BlockSpec((1, 3, n), lambda i: (i, 0, 0)),
        ],
        out_specs=pl.BlockSpec((1, 1, 128), lambda i: (i, 0, 0)),
        out_shape=jax.ShapeDtypeStruct((b, 1, 128), jnp.float32),
        scratch_shapes=[pltpu.VMEM((n, n), jnp.float32)],
        compiler_params=pltpu.CompilerParams(
            dimension_semantics=("parallel",),
            vmem_limit_bytes=100 * 1024 * 1024,
        ),
        name="sinkhorn_emd",
    )(pc1, pc2t)
    return jnp.sum(per_batch[:, 0, 0])
